# baseline (speedup 1.0000x reference)
import functools
import jax
import jax.numpy as jnp
from jax import lax
from jax.experimental import pallas as pl
from jax.experimental.pallas import tpu as pltpu
from jax.experimental.pallas import tpu_sc as plsc

_N = 10000
_E = 320000
_D = 128
_NE = 4
_NP = 10240
_NTILES = 16
_CH = 88
_NCH = 232
_NCH2 = _NCH // 2
_EPAD = _NTILES * _NCH * _CH
_DCH = 512
_NDCH = 40
_DPAD = _NTILES * _NDCH * _DCH
_BN = 256
_STRIPE = _NP // _NTILES


def _mm2_body(h_ref, w1_ref, w2_ref, b_ref, o1_ref, o2_ref):
    h = h_ref[...]
    o1_ref[...] = jnp.dot(h, w1_ref[...], preferred_element_type=jnp.float32)
    o2_ref[...] = (jnp.dot(h, w2_ref[...], preferred_element_type=jnp.float32)
                   + b_ref[...])


def _matmul2(h, w1, w2, b):
    din = h.shape[1]
    m1 = w1.shape[1]
    m2 = w2.shape[1]
    return pl.pallas_call(
        _mm2_body,
        grid=(_NP // _BN,),
        in_specs=[
            pl.BlockSpec((_BN, din), lambda i: (i, 0)),
            pl.BlockSpec((din, m1), lambda i: (0, 0)),
            pl.BlockSpec((din, m2), lambda i: (0, 0)),
            pl.BlockSpec((1, m2), lambda i: (0, 0)),
        ],
        out_specs=[
            pl.BlockSpec((_BN, m1), lambda i: (i, 0)),
            pl.BlockSpec((_BN, m2), lambda i: (i, 0)),
        ],
        out_shape=[
            jax.ShapeDtypeStruct((_NP, m1), jnp.float32),
            jax.ShapeDtypeStruct((_NP, m2), jnp.float32),
        ],
    )(h, w1, w2, b.reshape(1, m2))


def _mm2s_body(h_ref, w1_ref, w2_ref, b_ref, o1_ref, o2_ref):
    h = h_ref[...]
    o1_ref[0] = jnp.dot(h, w1_ref[0], preferred_element_type=jnp.float32)
    o2_ref[0] = (jnp.dot(h, w2_ref[0], preferred_element_type=jnp.float32)
                 + b_ref[0])


def _matmul2_stacked(h, wstack, wsh, bh):
    din = h.shape[1]
    m1 = wstack.shape[2]
    outh = wsh.shape[2]
    return pl.pallas_call(
        _mm2s_body,
        grid=(2, _NP // _BN),
        in_specs=[
            pl.BlockSpec((_BN, din), lambda c, i: (i, 0)),
            pl.BlockSpec((1, din, m1), lambda c, i: (c, 0, 0)),
            pl.BlockSpec((1, din, outh), lambda c, i: (c, 0, 0)),
            pl.BlockSpec((1, 1, outh), lambda c, i: (c, 0, 0)),
        ],
        out_specs=[
            pl.BlockSpec((1, _BN, m1), lambda c, i: (c, i, 0)),
            pl.BlockSpec((1, _BN, outh), lambda c, i: (c, i, 0)),
        ],
        out_shape=[
            jax.ShapeDtypeStruct((2, _NP, m1), jnp.float32),
            jax.ShapeDtypeStruct((2, _NP, outh), jnp.float32),
        ],
    )(h, wstack, wsh, bh)


def _combine_body(mode, has_res, refs):
    if has_res:
        agg_ref, deg_ref, hs_ref, res_ref, o_ref = refs
    else:
        agg_ref, deg_ref, hs_ref, o_ref = refs
    d2 = deg_ref[...]
    rdeg = 1.0 / jnp.maximum(d2[0:1, :] + d2[1:2, :], 1.0)
    rdeg = rdeg.reshape(_BN, 1)
    if mode == "split":
        a = (agg_ref[0] + agg_ref[1]) * rdeg + hs_ref[...]
        o = jnp.maximum(a, 0.0)
        if has_res:
            o = o + res_ref[...]
        o_ref[...] = o
    else:
        outh = agg_ref.shape[2]
        o0 = jnp.maximum(agg_ref[0] * rdeg + hs_ref[0], 0.0)
        o1 = jnp.maximum(agg_ref[1] * rdeg + hs_ref[1], 0.0)
        if has_res:
            o0 = o0 + res_ref[:, :outh]
            o1 = o1 + res_ref[:, outh:]
        o_ref[:, :outh] = o0
        o_ref[:, outh:] = o1


def _combine(mode, agg, deg, hs, res):
    outh = agg.shape[2]
    if mode == "split":
        out = hs.shape[1]
        hs_spec = pl.BlockSpec((_BN, out), lambda i: (i, 0))
    else:
        out = 2 * outh
        hs_spec = pl.BlockSpec((2, _BN, outh), lambda i: (0, i, 0))
    specs = [
        pl.BlockSpec((2, _BN, outh), lambda i: (0, i, 0)),
        pl.BlockSpec((2, _BN), lambda i: (0, i)),
        hs_spec,
    ]
    args = [agg, deg, hs]
    if res is not None:
        specs.append(pl.BlockSpec((_BN, out), lambda i: (i, 0)))
        args.append(res)
    body = functools.partial(_combine_body, mode, res is not None)
    return pl.pallas_call(
        lambda *refs: body(refs),
        grid=(_NP // _BN,),
        in_specs=specs,
        out_specs=pl.BlockSpec((_BN, out), lambda i: (i, 0)),
        out_shape=jax.ShapeDtypeStruct((_NP, out), jnp.float32),
    )(*args)


_MESH = plsc.VectorSubcoreMesh(core_axis_name="c", subcore_axis_name="s")


@functools.partial(jax.jit, static_argnums=(4,))
def _sc_scatter(ht, gidx, sdst, zeros, split):
    nch = _NCH2 if split else _NCH

    def body(ht_hbm, gidx_hbm, sdst_hbm, zeros_hbm, out_hbm, *scr):
        gis = scr[0:4]
        sds = scr[4:8]
        rows = scr[8:12]
        agg_s = scr[12]
        sis = scr[13:17]
        sgs = scr[17:21]
        sss = scr[21:25]
        cid = lax.axis_index("c")
        sid = lax.axis_index("s")
        row0 = sid * _STRIPE
        pltpu.sync_copy(zeros_hbm.at[pl.ds(row0, _STRIPE)],
                        agg_s.at[pl.ds(row0, _STRIPE)])
        plsc.subcore_barrier()
        if split:
            table = ht_hbm
            base = cid * _NCH2
        else:
            table = ht_hbm.at[cid]
            base = 0

        def idx_start(j, s):
            pltpu.async_copy(gidx_hbm.at[sid, base + j], gis[s], sis[s])
            pltpu.async_copy(sdst_hbm.at[sid, base + j], sds[s], sis[s])

        def idx_wait(j, s):
            pltpu.make_async_copy(gidx_hbm.at[sid, base + j], gis[s],
                                  sis[s]).wait()
            pltpu.make_async_copy(sdst_hbm.at[sid, base + j], sds[s],
                                  sis[s]).wait()

        def gather_start(s):
            pltpu.async_copy(table.at[gis[s]], rows[s], sgs[s])

        def gather_wait(s):
            pltpu.make_async_copy(table.at[gis[s]], rows[s], sgs[s]).wait()

        def scat_start(s):
            pltpu.async_copy(rows[s], agg_s.at[sds[s]], sss[s], add=True)

        def scat_drain(s):
            pltpu.make_async_copy(rows[s], agg_s.at[sds[s]], sss[s]).wait()

        def emit(t, u, prefetch):
            idx_wait(t, u)
            gather_start(u)
            gather_wait((u - 1) % 4)
            scat_start((u - 1) % 4)
            scat_drain((u - 2) % 4)
            if prefetch:
                idx_start(t + 2, (u - 2) % 4)

        idx_start(0, 0)
        idx_start(1, 1)
        idx_wait(0, 0)
        gather_start(0)
        idx_start(2, 2)
        idx_wait(1, 1)
        gather_start(1)
        gather_wait(0)
        scat_start(0)
        idx_start(3, 3)

        def quad(k, carry):
            t0 = 2 + k * 4
            emit(t0, 2, True)
            emit(t0 + 1, 3, True)
            emit(t0 + 2, 0, True)
            emit(t0 + 3, 1, True)
            return carry

        lax.fori_loop(0, (nch - 4) // 4, quad, 0)
        emit(nch - 2, 2, False)
        emit(nch - 1, 3, False)
        gather_wait(3)
        scat_start(3)
        scat_drain(2)
        scat_drain(3)
        plsc.subcore_barrier()
        pltpu.sync_copy(agg_s.at[pl.ds(row0, _STRIPE)],
                        out_hbm.at[cid, pl.ds(row0, _STRIPE)])

    return pl.kernel(
        body,
        out_type=jax.ShapeDtypeStruct((2, _NP, 128), jnp.float32),
        mesh=_MESH,
        scratch_types=(
            [pltpu.VMEM((_CH,), jnp.int32)] * 8
            + [pltpu.VMEM((_CH, 128), jnp.float32)] * 4
            + [pltpu.VMEM_SHARED((_NP, 128), jnp.float32)]
            + [pltpu.SemaphoreType.DMA] * 12
        ),
    )(ht, gidx, sdst, zeros)


@jax.jit
def _sc_degree(sdst, ones, zeros1):
    def body(sdst_hbm, ones_hbm, zeros_hbm, out_hbm, sdst_v, ones_v, deg_s, sem):
        cid = lax.axis_index("c")
        sid = lax.axis_index("s")
        pltpu.sync_copy(ones_hbm, ones_v)
        row0 = sid * _STRIPE
        pltpu.sync_copy(zeros_hbm.at[pl.ds(row0, _STRIPE)],
                        deg_s.at[pl.ds(row0, _STRIPE)])
        plsc.subcore_barrier()
        base = cid * (_NDCH // 2)

        def step(j, carry):
            pltpu.sync_copy(sdst_hbm.at[sid, base + j], sdst_v)
            pltpu.sync_copy(ones_v, deg_s.at[sdst_v], add=True)
            return carry

        lax.fori_loop(0, _NDCH // 2, step, 0)
        plsc.subcore_barrier()
        pltpu.sync_copy(deg_s.at[pl.ds(row0, _STRIPE)],
                        out_hbm.at[cid, pl.ds(row0, _STRIPE)])

    return pl.kernel(
        body,
        out_type=jax.ShapeDtypeStruct((2, _NP), jnp.float32),
        mesh=_MESH,
        scratch_types=[
            pltpu.VMEM((_DCH,), jnp.int32),
            pltpu.VMEM((_DCH,), jnp.float32),
            pltpu.VMEM_SHARED((_NP,), jnp.float32),
            pltpu.SemaphoreType.DMA,
        ],
    )(sdst, ones, zeros1)


def _wstack(w):
    ne, din, out = w.shape
    outh = out // 2
    return (w.reshape(ne, din, 2, outh)
             .transpose(2, 1, 0, 3)
             .reshape(2, din, ne * outh))


def _wflat(w):
    ne, din, out = w.shape
    return w.transpose(1, 0, 2).reshape(din, ne * out)


def _block(h, w, ws, b, gidx, sdst, deg, zeros, res):
    out = w.shape[2]
    if out == 128:
        ht, hs = _matmul2(h, _wflat(w), ws, b)
        ht = ht.reshape(_NP * _NE, 128)
        agg = _sc_scatter(ht, gidx, sdst, zeros, True)
        return _combine("split", agg, deg, hs, res)
    else:
        outh = out // 2
        din = ws.shape[0]
        wsh = ws.reshape(din, 2, outh).transpose(1, 0, 2)
        bh = b.reshape(2, 1, outh)
        ht, hs2 = _matmul2_stacked(h, _wstack(w), wsh, bh)
        ht = ht.reshape(2, _NP * _NE, 128)
        agg = _sc_scatter(ht, gidx, sdst, zeros, False)
        return _combine("cols", agg, deg, hs2, res)


def kernel(x, edge_index, edge_type, W1, Ws1, b1, W2, Ws2, b2,
           W3, Ws3, b3, W4, Ws4, b4, W5, Ws5, b5):
    src = edge_index[0]
    dst = edge_index[1]
    pad = _EPAD - _E
    fake_g = (jnp.arange(pad, dtype=jnp.int32) * 8) % (_N * _NE)
    fake_d = _N + jnp.arange(pad, dtype=jnp.int32) % (_NP - _N)
    gidx = jnp.concatenate(
        [src * _NE + edge_type, fake_g]
    ).reshape(_NTILES, _NCH, _CH)
    sdst = jnp.concatenate(
        [dst, fake_d]
    ).reshape(_NTILES, _NCH, _CH)

    pad_d = _DPAD - _E
    sdst_d = jnp.concatenate(
        [dst, _N + jnp.arange(pad_d, dtype=jnp.int32) % (_NP - _N)]
    ).reshape(_NTILES, _NDCH, _DCH)

    zeros = jnp.zeros((_NP, 128), jnp.float32)
    zeros1 = jnp.zeros((_NP,), jnp.float32)
    ones = jnp.ones((_DCH,), jnp.float32)
    deg = _sc_degree(sdst_d, ones, zeros1)

    hp = jnp.pad(x, ((0, _NP - _N), (0, 0)))
    h1 = _block(hp, W1, Ws1, b1, gidx, sdst, deg, zeros, hp)
    h2 = _block(h1, W2, Ws2, b2, gidx, sdst, deg, zeros, h1)
    hc = jnp.concatenate([hp, h2], axis=1)
    h3 = _block(hc, W3, Ws3, b3, gidx, sdst, deg, zeros, hc)
    h4 = _block(h3, W4, Ws4, b4, gidx, sdst, deg, zeros, h3)
    hc2 = jnp.concatenate([h2, h4], axis=1)
    h5 = _block(hc2, W5, Ws5, b5, gidx, sdst, deg, zeros, None)
    return h5[:_N]


# speedup vs baseline: 1.0032x; 1.0032x over previous
import functools
import jax
import jax.numpy as jnp
from jax import lax
from jax.experimental import pallas as pl
from jax.experimental.pallas import tpu as pltpu
from jax.experimental.pallas import tpu_sc as plsc

_N = 10000
_E = 320000
_D = 128
_NE = 4
_NP = 10240
_NTILES = 16
_CH = 88
_NCH = 232
_NCH2 = _NCH // 2
_EPAD = _NTILES * _NCH * _CH
_DCH = 512
_NDCH = 40
_DPAD = _NTILES * _NDCH * _DCH
_BN = 256
_STRIPE = _NP // _NTILES


def _mm2_body(h_ref, w1_ref, w2_ref, b_ref, o1_ref, o2_ref):
    h = h_ref[...].astype(jnp.bfloat16)
    o1_ref[...] = jnp.dot(h, w1_ref[...], preferred_element_type=jnp.float32)
    o2_ref[...] = (jnp.dot(h, w2_ref[...], preferred_element_type=jnp.float32)
                   + b_ref[...])


def _matmul2(h, w1, w2, b):
    din = h.shape[1]
    m1 = w1.shape[1]
    m2 = w2.shape[1]
    return pl.pallas_call(
        _mm2_body,
        grid=(_NP // _BN,),
        in_specs=[
            pl.BlockSpec((_BN, din), lambda i: (i, 0)),
            pl.BlockSpec((din, m1), lambda i: (0, 0)),
            pl.BlockSpec((din, m2), lambda i: (0, 0)),
            pl.BlockSpec((1, m2), lambda i: (0, 0)),
        ],
        out_specs=[
            pl.BlockSpec((_BN, m1), lambda i: (i, 0)),
            pl.BlockSpec((_BN, m2), lambda i: (i, 0)),
        ],
        out_shape=[
            jax.ShapeDtypeStruct((_NP, m1), jnp.float32),
            jax.ShapeDtypeStruct((_NP, m2), jnp.float32),
        ],
    )(h, w1, w2, b.reshape(1, m2))


def _mm2s_body(h_ref, w1_ref, w2_ref, b_ref, o1_ref, o2_ref):
    h = h_ref[...].astype(jnp.bfloat16)
    o1_ref[0] = jnp.dot(h, w1_ref[0], preferred_element_type=jnp.float32)
    o2_ref[0] = (jnp.dot(h, w2_ref[0], preferred_element_type=jnp.float32)
                 + b_ref[0])


def _matmul2_stacked(h, wstack, wsh, bh):
    din = h.shape[1]
    m1 = wstack.shape[2]
    outh = wsh.shape[2]
    return pl.pallas_call(
        _mm2s_body,
        grid=(2, _NP // _BN),
        in_specs=[
            pl.BlockSpec((_BN, din), lambda c, i: (i, 0)),
            pl.BlockSpec((1, din, m1), lambda c, i: (c, 0, 0)),
            pl.BlockSpec((1, din, outh), lambda c, i: (c, 0, 0)),
            pl.BlockSpec((1, 1, outh), lambda c, i: (c, 0, 0)),
        ],
        out_specs=[
            pl.BlockSpec((1, _BN, m1), lambda c, i: (c, i, 0)),
            pl.BlockSpec((1, _BN, outh), lambda c, i: (c, i, 0)),
        ],
        out_shape=[
            jax.ShapeDtypeStruct((2, _NP, m1), jnp.float32),
            jax.ShapeDtypeStruct((2, _NP, outh), jnp.float32),
        ],
    )(h, wstack, wsh, bh)


def _combine_body(mode, has_res, refs):
    if has_res:
        agg_ref, deg_ref, hs_ref, res_ref, o_ref = refs
    else:
        agg_ref, deg_ref, hs_ref, o_ref = refs
    d2 = deg_ref[...]
    rdeg = 1.0 / jnp.maximum(d2[0:1, :] + d2[1:2, :], 1.0)
    rdeg = rdeg.reshape(_BN, 1)
    if mode == "split":
        a = (agg_ref[0] + agg_ref[1]) * rdeg + hs_ref[...]
        o = jnp.maximum(a, 0.0)
        if has_res:
            o = o + res_ref[...]
        o_ref[...] = o
    else:
        outh = agg_ref.shape[2]
        o0 = jnp.maximum(agg_ref[0] * rdeg + hs_ref[0], 0.0)
        o1 = jnp.maximum(agg_ref[1] * rdeg + hs_ref[1], 0.0)
        if has_res:
            o0 = o0 + res_ref[:, :outh]
            o1 = o1 + res_ref[:, outh:]
        o_ref[:, :outh] = o0
        o_ref[:, outh:] = o1


def _combine(mode, agg, deg, hs, res):
    outh = agg.shape[2]
    if mode == "split":
        out = hs.shape[1]
        hs_spec = pl.BlockSpec((_BN, out), lambda i: (i, 0))
    else:
        out = 2 * outh
        hs_spec = pl.BlockSpec((2, _BN, outh), lambda i: (0, i, 0))
    specs = [
        pl.BlockSpec((2, _BN, outh), lambda i: (0, i, 0)),
        pl.BlockSpec((2, _BN), lambda i: (0, i)),
        hs_spec,
    ]
    args = [agg, deg, hs]
    if res is not None:
        specs.append(pl.BlockSpec((_BN, out), lambda i: (i, 0)))
        args.append(res)
    body = functools.partial(_combine_body, mode, res is not None)
    return pl.pallas_call(
        lambda *refs: body(refs),
        grid=(_NP // _BN,),
        in_specs=specs,
        out_specs=pl.BlockSpec((_BN, out), lambda i: (i, 0)),
        out_shape=jax.ShapeDtypeStruct((_NP, out), jnp.float32),
    )(*args)


_MESH = plsc.VectorSubcoreMesh(core_axis_name="c", subcore_axis_name="s")


@functools.partial(jax.jit, static_argnums=(4,))
def _sc_scatter(ht, gidx, sdst, zeros, split):
    nch = _NCH2 if split else _NCH

    def body(ht_hbm, gidx_hbm, sdst_hbm, zeros_hbm, out_hbm, *scr):
        gis = scr[0:4]
        sds = scr[4:8]
        rows = scr[8:12]
        agg_s = scr[12]
        sis = scr[13:17]
        sgs = scr[17:21]
        sss = scr[21:25]
        cid = lax.axis_index("c")
        sid = lax.axis_index("s")
        row0 = sid * _STRIPE
        pltpu.sync_copy(zeros_hbm.at[pl.ds(row0, _STRIPE)],
                        agg_s.at[pl.ds(row0, _STRIPE)])
        plsc.subcore_barrier()
        if split:
            table = ht_hbm
            base = cid * _NCH2
        else:
            table = ht_hbm.at[cid]
            base = 0

        def idx_start(j, s):
            pltpu.async_copy(gidx_hbm.at[sid, base + j], gis[s], sis[s])
            pltpu.async_copy(sdst_hbm.at[sid, base + j], sds[s], sis[s])

        def idx_wait(j, s):
            pltpu.make_async_copy(gidx_hbm.at[sid, base + j], gis[s],
                                  sis[s]).wait()
            pltpu.make_async_copy(sdst_hbm.at[sid, base + j], sds[s],
                                  sis[s]).wait()

        def gather_start(s):
            pltpu.async_copy(table.at[gis[s]], rows[s], sgs[s])

        def gather_wait(s):
            pltpu.make_async_copy(table.at[gis[s]], rows[s], sgs[s]).wait()

        def scat_start(s):
            pltpu.async_copy(rows[s], agg_s.at[sds[s]], sss[s], add=True)

        def scat_drain(s):
            pltpu.make_async_copy(rows[s], agg_s.at[sds[s]], sss[s]).wait()

        def emit(t, u, prefetch):
            idx_wait(t, u)
            gather_start(u)
            gather_wait((u - 1) % 4)
            scat_start((u - 1) % 4)
            scat_drain((u - 2) % 4)
            if prefetch:
                idx_start(t + 2, (u - 2) % 4)

        idx_start(0, 0)
        idx_start(1, 1)
        idx_wait(0, 0)
        gather_start(0)
        idx_start(2, 2)
        idx_wait(1, 1)
        gather_start(1)
        gather_wait(0)
        scat_start(0)
        idx_start(3, 3)

        def quad(k, carry):
            t0 = 2 + k * 4
            emit(t0, 2, True)
            emit(t0 + 1, 3, True)
            emit(t0 + 2, 0, True)
            emit(t0 + 3, 1, True)
            return carry

        lax.fori_loop(0, (nch - 4) // 4, quad, 0)
        emit(nch - 2, 2, False)
        emit(nch - 1, 3, False)
        gather_wait(3)
        scat_start(3)
        scat_drain(2)
        scat_drain(3)
        plsc.subcore_barrier()
        pltpu.sync_copy(agg_s.at[pl.ds(row0, _STRIPE)],
                        out_hbm.at[cid, pl.ds(row0, _STRIPE)])

    return pl.kernel(
        body,
        out_type=jax.ShapeDtypeStruct((2, _NP, 128), jnp.float32),
        mesh=_MESH,
        scratch_types=(
            [pltpu.VMEM((_CH,), jnp.int32)] * 8
            + [pltpu.VMEM((_CH, 128), jnp.float32)] * 4
            + [pltpu.VMEM_SHARED((_NP, 128), jnp.float32)]
            + [pltpu.SemaphoreType.DMA] * 12
        ),
    )(ht, gidx, sdst, zeros)


@jax.jit
def _sc_degree(sdst, ones, zeros1):
    def body(sdst_hbm, ones_hbm, zeros_hbm, out_hbm, sdst_v, ones_v, deg_s, sem):
        cid = lax.axis_index("c")
        sid = lax.axis_index("s")
        pltpu.sync_copy(ones_hbm, ones_v)
        row0 = sid * _STRIPE
        pltpu.sync_copy(zeros_hbm.at[pl.ds(row0, _STRIPE)],
                        deg_s.at[pl.ds(row0, _STRIPE)])
        plsc.subcore_barrier()
        base = cid * (_NDCH // 2)

        def step(j, carry):
            pltpu.sync_copy(sdst_hbm.at[sid, base + j], sdst_v)
            pltpu.sync_copy(ones_v, deg_s.at[sdst_v], add=True)
            return carry

        lax.fori_loop(0, _NDCH // 2, step, 0)
        plsc.subcore_barrier()
        pltpu.sync_copy(deg_s.at[pl.ds(row0, _STRIPE)],
                        out_hbm.at[cid, pl.ds(row0, _STRIPE)])

    return pl.kernel(
        body,
        out_type=jax.ShapeDtypeStruct((2, _NP), jnp.float32),
        mesh=_MESH,
        scratch_types=[
            pltpu.VMEM((_DCH,), jnp.int32),
            pltpu.VMEM((_DCH,), jnp.float32),
            pltpu.VMEM_SHARED((_NP,), jnp.float32),
            pltpu.SemaphoreType.DMA,
        ],
    )(sdst, ones, zeros1)


def _wstack(w):
    ne, din, out = w.shape
    outh = out // 2
    return (w.reshape(ne, din, 2, outh)
             .transpose(2, 1, 0, 3)
             .reshape(2, din, ne * outh))


def _wflat(w):
    ne, din, out = w.shape
    return w.transpose(1, 0, 2).reshape(din, ne * out)


def _block(h, w, ws, b, gidx, sdst, deg, zeros, res):
    out = w.shape[2]
    if out == 128:
        ht, hs = _matmul2(h, _wflat(w).astype(jnp.bfloat16),
                          ws.astype(jnp.bfloat16), b)
        ht = ht.reshape(_NP * _NE, 128)
        agg = _sc_scatter(ht, gidx, sdst, zeros, True)
        return _combine("split", agg, deg, hs, res)
    else:
        outh = out // 2
        din = ws.shape[0]
        wsh = ws.reshape(din, 2, outh).transpose(1, 0, 2)
        bh = b.reshape(2, 1, outh)
        ht, hs2 = _matmul2_stacked(h, _wstack(w).astype(jnp.bfloat16),
                                   wsh.astype(jnp.bfloat16), bh)
        ht = ht.reshape(2, _NP * _NE, 128)
        agg = _sc_scatter(ht, gidx, sdst, zeros, False)
        return _combine("cols", agg, deg, hs2, res)


def kernel(x, edge_index, edge_type, W1, Ws1, b1, W2, Ws2, b2,
           W3, Ws3, b3, W4, Ws4, b4, W5, Ws5, b5):
    src = edge_index[0]
    dst = edge_index[1]
    pad = _EPAD - _E
    fake_g = (jnp.arange(pad, dtype=jnp.int32) * 8) % (_N * _NE)
    fake_d = _N + jnp.arange(pad, dtype=jnp.int32) % (_NP - _N)
    gidx = jnp.concatenate(
        [src * _NE + edge_type, fake_g]
    ).reshape(_NTILES, _NCH, _CH)
    sdst = jnp.concatenate(
        [dst, fake_d]
    ).reshape(_NTILES, _NCH, _CH)

    pad_d = _DPAD - _E
    sdst_d = jnp.concatenate(
        [dst, _N + jnp.arange(pad_d, dtype=jnp.int32) % (_NP - _N)]
    ).reshape(_NTILES, _NDCH, _DCH)

    zeros = jnp.zeros((_NP, 128), jnp.float32)
    zeros1 = jnp.zeros((_NP,), jnp.float32)
    ones = jnp.ones((_DCH,), jnp.float32)
    deg = _sc_degree(sdst_d, ones, zeros1)

    hp = jnp.pad(x, ((0, _NP - _N), (0, 0)))
    h1 = _block(hp, W1, Ws1, b1, gidx, sdst, deg, zeros, hp)
    h2 = _block(h1, W2, Ws2, b2, gidx, sdst, deg, zeros, h1)
    hc = jnp.concatenate([hp, h2], axis=1)
    h3 = _block(hc, W3, Ws3, b3, gidx, sdst, deg, zeros, hc)
    h4 = _block(h3, W4, Ws4, b4, gidx, sdst, deg, zeros, h3)
    hc2 = jnp.concatenate([h2, h4], axis=1)
    h5 = _block(hc2, W5, Ws5, b5, gidx, sdst, deg, zeros, None)
    return h5[:_N]
